# baseline (speedup 1.0000x reference)
import functools

import jax
import jax.numpy as jnp
from jax import lax
from jax.experimental import pallas as pl
from jax.experimental.pallas import tpu as pltpu
from jax.experimental.pallas import tpu_sc as plsc

BATCH = 4096
NW = 32
LANES = 16
CHUNK = 8


def kernel(x, pos_table):
    batch, maxlen, dim = x.shape
    rows = maxlen * dim
    rows_per_w = rows // NW
    xt = jnp.transpose(x, (1, 2, 0)).reshape(rows, batch)
    pos_b = jnp.broadcast_to(pos_table.reshape(rows, 1), (rows, LANES))

    mesh = plsc.VectorSubcoreMesh(core_axis_name="c", subcore_axis_name="s")

    @functools.partial(
        pl.kernel,
        mesh=mesh,
        out_type=jax.ShapeDtypeStruct((rows, batch), jnp.float32),
        scratch_types=[
            pltpu.VMEM((CHUNK, LANES), jnp.float32),
            pltpu.VMEM((CHUNK, BATCH), jnp.float32),
            pltpu.SemaphoreType.DMA,
        ],
    )
    def sc_add(x_hbm, posb_hbm, out_hbm, posb_v, buf_v, sem):
        wid = lax.axis_index("s") * 2 + lax.axis_index("c")
        base = wid * rows_per_w

        def chunk_body(cj, _):
            r0 = base + cj * CHUNK
            pltpu.sync_copy(posb_hbm.at[pl.ds(r0, CHUNK)], posb_v)
            pltpu.async_copy(x_hbm.at[pl.ds(r0, CHUNK)], buf_v, sem).wait()

            def row_body(j, _):
                pos16 = posb_v[j, :]

                def vec_body(k, _):
                    sl = pl.ds(k * LANES, LANES)
                    buf_v[j, sl] = buf_v[j, sl] + pos16
                    return 0

                return lax.fori_loop(0, BATCH // LANES, vec_body, 0)

            lax.fori_loop(0, CHUNK, row_body, 0)
            pltpu.async_copy(buf_v, out_hbm.at[pl.ds(r0, CHUNK)], sem).wait()
            return 0

        lax.fori_loop(0, rows_per_w // CHUNK, chunk_body, 0)

    out = sc_add(xt, pos_b)
    return out.reshape(maxlen, dim, batch).transpose(2, 0, 1)


# speedup vs baseline: 5.7769x; 5.7769x over previous
import jax
import jax.numpy as jnp
from jax.experimental import pallas as pl
from jax.experimental.pallas import tpu as pltpu

T_BLOCK = 10


def _add_kernel(x_ref, pos_ref, out_ref):
    i = pl.program_id(0)
    pos = pos_ref[pl.ds(i * T_BLOCK, T_BLOCK), :]
    out_ref[...] = x_ref[...] + pos[:, :, None]


def kernel(x, pos_table):
    batch, maxlen, dim = x.shape
    xt = jnp.transpose(x, (1, 2, 0))
    grid = (maxlen // T_BLOCK,)
    out = pl.pallas_call(
        _add_kernel,
        grid=grid,
        in_specs=[
            pl.BlockSpec((T_BLOCK, dim, batch), lambda i: (i, 0, 0)),
            pl.BlockSpec((maxlen, dim), lambda i: (0, 0)),
        ],
        out_specs=pl.BlockSpec((T_BLOCK, dim, batch), lambda i: (i, 0, 0)),
        out_shape=jax.ShapeDtypeStruct((maxlen, dim, batch), x.dtype),
        compiler_params=pltpu.CompilerParams(
            dimension_semantics=("parallel",)),
    )(xt, pos_table)
    return out.transpose(2, 0, 1)
